# baseline (speedup 1.0000x reference)
import jax
import jax.numpy as jnp
from jax import lax
from jax.experimental import pallas as pl
from jax.experimental.pallas import tpu as pltpu
from jax.experimental.pallas import tpu_sc as plsc

_D = 64
_NC, _NS = 2, 16
_NW = _NC * _NS
_CHUNK = 128
_TILE = 2048
_EPS = 1e-5


def _sc_gather_body(uidx, iidx, gu, gi, mu, mi,
                    out_gu, out_gi, out_mu, out_mi,
                    idxu_v, idxi_v, buf_a, buf_b, sem_a, sem_b):
    cpw = idxu_v.shape[0]
    wid = lax.axis_index("s") * _NC + lax.axis_index("c")
    row0 = wid * cpw
    pltpu.sync_copy(uidx.at[pl.ds(row0, cpw)], idxu_v)
    pltpu.sync_copy(iidx.at[pl.ds(row0, cpw)], idxi_v)

    plan = ((gu, idxu_v, out_gu), (gi, idxi_v, out_gi),
            (mu, idxu_v, out_mu), (mi, idxi_v, out_mi))
    bufs = (buf_a, buf_b)
    sems = (sem_a, sem_b)

    def fire(t):
        tab, idx_v, _ = plan[t]
        return [pltpu.async_copy(tab.at[idx_v.at[j]], bufs[t % 2].at[j],
                                 sems[t % 2])
                for j in range(cpw)]

    def drain_and_store(t, copies):
        for c in copies[t]:
            c.wait()
        pltpu.sync_copy(bufs[t % 2], plan[t][2].at[pl.ds(row0, cpw)])

    copies = {0: fire(0), 1: fire(1)}
    drain_and_store(0, copies)
    copies[2] = fire(2)
    drain_and_store(1, copies)
    copies[3] = fire(3)
    drain_and_store(2, copies)
    drain_and_store(3, copies)


_F32 = jnp.float32
_HI = lax.Precision.HIGHEST


def _accum_stats(i, y, st_ref):
    ps = jnp.sum(y, axis=0, keepdims=True)
    pq = jnp.sum(y * y, axis=0, keepdims=True)
    part = jnp.concatenate([ps, pq], axis=0)

    @pl.when(i == 0)
    def _():
        st_ref[...] = part

    @pl.when(i > 0)
    def _():
        st_ref[...] += part


def _bn_from_stats(st_ref, n, g, be, y):
    mean = st_ref[0:1, :] * (1.0 / n)
    var = st_ref[1:2, :] * (1.0 / n) - mean * mean
    return jnp.maximum((y - mean) * lax.rsqrt(var + _EPS) * g + be, 0.0)


def _stage1_body(xu, xi, w1ut, w1it, b1, y1_ref, st1_ref):
    i = pl.program_id(0)
    y1 = (jnp.dot(xu[...], w1ut[...], preferred_element_type=_F32, precision=_HI)
          + jnp.dot(xi[...], w1it[...], preferred_element_type=_F32, precision=_HI)
          + b1[...])
    y1_ref[...] = y1
    _accum_stats(i, y1, st1_ref)


def _stage2_body(y1, st1, g1, be1, w2t, b2, y2_ref, st2_ref, *, n):
    i = pl.program_id(0)
    h1 = _bn_from_stats(st1, n, g1[...], be1[...], y1[...])
    y2 = jnp.dot(h1, w2t[...], preferred_element_type=_F32, precision=_HI) + b2[...]
    y2_ref[...] = y2
    _accum_stats(i, y2, st2_ref)


def _stage3_body(y2, st2, g2, be2, ug, ig, wg, wh, bo, out_ref, *, n):
    h2 = _bn_from_stats(st2, n, g2[...], be2[...], y2[...])
    xg = ug[...] * ig[...]
    s = (jnp.sum(xg * wg[...], axis=1, keepdims=True)
         + jnp.sum(h2 * wh[...], axis=1, keepdims=True) + bo[...])
    out_ref[...] = 1.0 / (1.0 + jnp.exp(-s))


def kernel(user_idx, item_idx, gmf_user, gmf_item, mlp_user, mlp_item,
           W1, b1, g1, be1, W2, b2, g2, be2, Wout, bout):
    B = user_idx.shape[0]
    nchunks = B // _CHUNK
    cpw = nchunks // _NW
    uidx = user_idx.astype(jnp.int32).reshape(nchunks, _CHUNK)
    iidx = item_idx.astype(jnp.int32).reshape(nchunks, _CHUNK)

    out_sh = jax.ShapeDtypeStruct((nchunks, _CHUNK, _D), jnp.float32)
    sc_gather = pl.kernel(
        _sc_gather_body,
        out_type=(out_sh,) * 4,
        mesh=plsc.VectorSubcoreMesh(core_axis_name="c", subcore_axis_name="s",
                                    num_cores=_NC, num_subcores=_NS),
        compiler_params=pltpu.CompilerParams(use_tc_tiling_on_sc=False),
        scratch_types=[
            pltpu.VMEM((cpw, _CHUNK), jnp.int32),
            pltpu.VMEM((cpw, _CHUNK), jnp.int32),
            pltpu.VMEM((cpw, _CHUNK, _D), jnp.float32),
            pltpu.VMEM((cpw, _CHUNK, _D), jnp.float32),
            pltpu.SemaphoreType.DMA,
            pltpu.SemaphoreType.DMA,
        ],
    )
    ug, ig, mu, mi = sc_gather(uidx, iidx, gmf_user, gmf_item,
                               mlp_user, mlp_item)
    ug = ug.reshape(B, _D)
    ig = ig.reshape(B, _D)
    mu = mu.reshape(B, _D)
    mi = mi.reshape(B, _D)

    H1 = W1.shape[0]
    H2 = W2.shape[0]
    tile = _TILE
    nt = B // tile
    w1t = W1.T
    f32 = jnp.float32

    def full(a):
        return pl.BlockSpec(a.shape, lambda i: (0,) * a.ndim)

    row_spec = lambda w: pl.BlockSpec((tile, w), lambda i: (i, 0))
    st_spec = lambda w: pl.BlockSpec((2, w), lambda i: (0, 0))

    b1r, g1r, be1r = (v.reshape(1, -1) for v in (b1, g1, be1))
    b2r, g2r, be2r = (v.reshape(1, -1) for v in (b2, g2, be2))

    y1, st1 = pl.pallas_call(
        _stage1_body,
        grid=(nt,),
        in_specs=[row_spec(_D), row_spec(_D),
                  full(w1t[:_D]), full(w1t[_D:]), full(b1r)],
        out_specs=[row_spec(H1), st_spec(H1)],
        out_shape=[jax.ShapeDtypeStruct((B, H1), f32),
                   jax.ShapeDtypeStruct((2, H1), f32)],
    )(mu, mi, w1t[:_D], w1t[_D:], b1r)

    import functools as _ft
    w2t = W2.T
    y2, st2 = pl.pallas_call(
        _ft.partial(_stage2_body, n=float(B)),
        grid=(nt,),
        in_specs=[row_spec(H1), st_spec(H1),
                  full(g1r), full(be1r), full(w2t), full(b2r)],
        out_specs=[row_spec(H2), st_spec(H2)],
        out_shape=[jax.ShapeDtypeStruct((B, H2), f32),
                   jax.ShapeDtypeStruct((2, H2), f32)],
    )(y1, st1, g1r, be1r, w2t, b2r)

    wg = Wout[:, :_D]
    wh = Wout[:, _D:]
    bor = bout.reshape(1, 1)
    out2 = pl.pallas_call(
        _ft.partial(_stage3_body, n=float(B)),
        grid=(nt,),
        in_specs=[row_spec(H2), st_spec(H2), full(g2r), full(be2r),
                  row_spec(_D), row_spec(_D), full(wg), full(wh), full(bor)],
        out_specs=pl.BlockSpec((tile, 1), lambda i: (i, 0)),
        out_shape=jax.ShapeDtypeStruct((B, 1), f32),
    )(y2, st2, g2r, be2r, ug, ig, wg, wh, bor)
    return out2.reshape(B)


# speedup vs baseline: 1.2476x; 1.2476x over previous
import functools

import jax
import jax.numpy as jnp
from jax import lax
from jax.experimental import pallas as pl
from jax.experimental.pallas import tpu as pltpu
from jax.experimental.pallas import tpu_sc as plsc

_D = 64
_NC, _NS = 2, 16
_NW = _NC * _NS
_CHUNK = 128
_TBLK = 1024
_TILE = 2048
_EPS = 1e-5
_F32 = jnp.float32
_HI = lax.Precision.HIGHEST


def _transcat_body(ga, ma, out_ref):
    tg = jnp.transpose(ga[...], (1, 0))
    tm = jnp.transpose(ma[...], (1, 0))
    out_ref[...] = jnp.concatenate([tg, tm], axis=1)


def _transcat(gt, mt):
    V = gt.shape[1]
    n = -(-V // _TBLK)
    return pl.pallas_call(
        _transcat_body,
        grid=(n,),
        in_specs=[pl.BlockSpec((_D, _TBLK), lambda i: (0, i)),
                  pl.BlockSpec((_D, _TBLK), lambda i: (0, i))],
        out_specs=pl.BlockSpec((_TBLK, 2 * _D), lambda i: (i, 0)),
        out_shape=jax.ShapeDtypeStruct((V, 2 * _D), _F32),
    )(gt, mt)


def _sc_gather_body(uidx, iidx, ucat, icat, out_u, out_i,
                    idxu_v, idxi_v, b0, b1, b2, b3, s0, s1, s2, s3):
    rpw = idxu_v.shape[0]
    cpt = rpw // _CHUNK
    wid = lax.axis_index("s") * _NC + lax.axis_index("c")
    base = wid * rpw
    pltpu.sync_copy(uidx.at[pl.ds(base, rpw)], idxu_v)
    pltpu.sync_copy(iidx.at[pl.ds(base, rpw)], idxi_v)

    bufs = (b0, b1, b2, b3)
    sems = (s0, s1, s2, s3)
    units = ([(ucat, idxu_v, out_u, j) for j in range(cpt)]
             + [(icat, idxi_v, out_i, j) for j in range(cpt)])
    nbuf = len(bufs)
    copies = {}

    def fire(t):
        tab, idxv, _, j = units[t]
        copies[t] = pltpu.async_copy(
            tab.at[idxv.at[pl.ds(j * _CHUNK, _CHUNK)]],
            bufs[t % nbuf], sems[t % nbuf])

    def drain(t):
        copies[t].wait()
        _, _, out, j = units[t]
        pltpu.sync_copy(bufs[t % nbuf], out.at[pl.ds(base + j * _CHUNK, _CHUNK)])

    for t in range(len(units)):
        if t >= nbuf:
            drain(t - nbuf)
        fire(t)
    for t in range(len(units) - nbuf, len(units)):
        drain(t)


def _accum_stats(i, y, st_ref):
    ps = jnp.sum(y, axis=0, keepdims=True)
    pq = jnp.sum(y * y, axis=0, keepdims=True)
    part = jnp.concatenate([ps, pq], axis=0)

    @pl.when(i == 0)
    def _():
        st_ref[...] = part

    @pl.when(i > 0)
    def _():
        st_ref[...] += part


def _bn_relu_from_stats(st_ref, n, g, be, y):
    mean = st_ref[0:1, :] * (1.0 / n)
    var = st_ref[1:2, :] * (1.0 / n) - mean * mean
    return jnp.maximum((y - mean) * lax.rsqrt(var + _EPS) * g + be, 0.0)


def _stage1_body(uc, ic, w1ut, w1it, b1, y1_ref, st1_ref, xg_ref):
    i = pl.program_id(0)
    ucv = uc[...]
    icv = ic[...]
    y1 = (jnp.dot(ucv[:, _D:], w1ut[...], preferred_element_type=_F32,
                  precision=_HI)
          + jnp.dot(icv[:, _D:], w1it[...], preferred_element_type=_F32,
                    precision=_HI)
          + b1[...])
    y1_ref[...] = y1
    xg_ref[...] = ucv[:, :_D] * icv[:, :_D]
    _accum_stats(i, y1, st1_ref)


def _stage2_body(y1, st1, g1, be1, w2t, b2, y2_ref, st2_ref, *, n):
    i = pl.program_id(0)
    h1 = _bn_relu_from_stats(st1, n, g1[...], be1[...], y1[...])
    y2 = jnp.dot(h1, w2t[...], preferred_element_type=_F32, precision=_HI) + b2[...]
    y2_ref[...] = y2
    _accum_stats(i, y2, st2_ref)


def _stage3_body(y2, st2, g2, be2, xg, wg, wh, bo, out_ref, *, n):
    h2 = _bn_relu_from_stats(st2, n, g2[...], be2[...], y2[...])
    s = (jnp.sum(xg[...] * wg[...], axis=1, keepdims=True)
         + jnp.sum(h2 * wh[...], axis=1, keepdims=True) + bo[...])
    out_ref[...] = 1.0 / (1.0 + jnp.exp(-s))


def kernel(user_idx, item_idx, gmf_user, gmf_item, mlp_user, mlp_item,
           W1, b1, g1, be1, W2, b2, g2, be2, Wout, bout):
    B = user_idx.shape[0]
    uidx = user_idx.astype(jnp.int32)
    iidx = item_idx.astype(jnp.int32)

    ucat = _transcat(gmf_user.T, mlp_user.T)
    icat = _transcat(gmf_item.T, mlp_item.T)

    rpw = B // _NW
    sc_gather = pl.kernel(
        _sc_gather_body,
        out_type=(jax.ShapeDtypeStruct((B, 2 * _D), _F32),) * 2,
        mesh=plsc.VectorSubcoreMesh(core_axis_name="c", subcore_axis_name="s",
                                    num_cores=_NC, num_subcores=_NS),
        scratch_types=(
            [pltpu.VMEM((rpw,), jnp.int32)] * 2
            + [pltpu.VMEM((_CHUNK, 2 * _D), _F32)] * 4
            + [pltpu.SemaphoreType.DMA] * 4
        ),
    )
    uc_g, ic_g = sc_gather(uidx, iidx, ucat, icat)

    H1 = W1.shape[0]
    H2 = W2.shape[0]
    tile = _TILE
    nt = B // tile
    w1t = W1.T
    w2t = W2.T

    def full(a):
        return pl.BlockSpec(a.shape, lambda i: (0,) * a.ndim)

    row_spec = lambda w: pl.BlockSpec((tile, w), lambda i: (i, 0))
    st_spec = lambda w: pl.BlockSpec((2, w), lambda i: (0, 0))

    b1r, g1r, be1r = (v.reshape(1, -1) for v in (b1, g1, be1))
    b2r, g2r, be2r = (v.reshape(1, -1) for v in (b2, g2, be2))

    y1, st1, xg = pl.pallas_call(
        _stage1_body,
        grid=(nt,),
        in_specs=[row_spec(2 * _D), row_spec(2 * _D),
                  full(w1t[:_D]), full(w1t[_D:]), full(b1r)],
        out_specs=[row_spec(H1), st_spec(H1), row_spec(_D)],
        out_shape=[jax.ShapeDtypeStruct((B, H1), _F32),
                   jax.ShapeDtypeStruct((2, H1), _F32),
                   jax.ShapeDtypeStruct((B, _D), _F32)],
    )(uc_g, ic_g, w1t[:_D], w1t[_D:], b1r)

    y2, st2 = pl.pallas_call(
        functools.partial(_stage2_body, n=float(B)),
        grid=(nt,),
        in_specs=[row_spec(H1), st_spec(H1),
                  full(g1r), full(be1r), full(w2t), full(b2r)],
        out_specs=[row_spec(H2), st_spec(H2)],
        out_shape=[jax.ShapeDtypeStruct((B, H2), _F32),
                   jax.ShapeDtypeStruct((2, H2), _F32)],
    )(y1, st1, g1r, be1r, w2t, b2r)

    wg = Wout[:, :_D]
    wh = Wout[:, _D:]
    bor = bout.reshape(1, 1)
    out2 = pl.pallas_call(
        functools.partial(_stage3_body, n=float(B)),
        grid=(nt,),
        in_specs=[row_spec(H2), st_spec(H2), full(g2r), full(be2r),
                  row_spec(_D), full(wg), full(wh), full(bor)],
        out_specs=pl.BlockSpec((tile, 1), lambda i: (i, 0)),
        out_shape=jax.ShapeDtypeStruct((B, 1), _F32),
    )(y2, st2, g2r, be2r, xg, wg, wh, bor)
    return out2.reshape(B)
